# Initial kernel scaffold; baseline (speedup 1.0000x reference)
import jax
import jax.numpy as jnp
from jax.experimental import pallas as pl


def kernel(x, edge_index, W, b):
    raise NotImplementedError("write your pallas kernel here")


# speedup vs baseline: 21.9378x; 21.9378x over previous
import functools

import jax
import jax.numpy as jnp
from jax import lax
from jax.experimental import pallas as pl
from jax.experimental.pallas import tpu as pltpu
from jax.experimental.pallas import tpu_sc as plsc

N = 10000
E = 320000
D = 128
NC = 2
NS = 16
NW = NC * NS
C = 128
CH = 80
CHP = 40
EPW = C * CH
EP = EPW * NW
PAD_ROWS = 240
ACC_ROWS = N + PAD_ROWS
ZSPAN = ACC_ROWS // NS
OSPAN = N // NS


def _sc_mesh():
    return plsc.VectorSubcoreMesh(core_axis_name="c", subcore_axis_name="s")


def _deg_body(col_hbm, out_hbm, colv, ones_v, zero_v, acc):
    c = lax.axis_index("c")
    s = lax.axis_index("s")
    w = c * NS + s

    def fill_z(i, carry):
        zero_v[pl.ds(i * 16, 16)] = jnp.zeros((16,), jnp.float32)
        return carry

    lax.fori_loop(0, ZSPAN // 16, fill_z, 0)

    def fill_o(i, carry):
        ones_v[pl.ds(i * 16, 16)] = jnp.ones((16,), jnp.float32)
        return carry

    lax.fori_loop(0, C // 16, fill_o, 0)

    pltpu.sync_copy(zero_v, acc.at[pl.ds(s * ZSPAN, ZSPAN)])
    plsc.subcore_barrier()

    pltpu.sync_copy(col_hbm.at[w], colv)

    def chunk(j, carry):
        pltpu.sync_copy(ones_v, acc.at[colv.at[j]], add=True)
        return carry

    lax.fori_loop(0, CH, chunk, 0)
    plsc.subcore_barrier()

    pltpu.sync_copy(acc.at[pl.ds(s * ZSPAN, ZSPAN)], zero_v)
    pltpu.sync_copy(zero_v, out_hbm.at[c, pl.ds(s * ZSPAN, ZSPAN)])


_deg_call = pl.kernel(
    _deg_body,
    out_type=jax.ShapeDtypeStruct((NC, ACC_ROWS), jnp.float32),
    mesh=_sc_mesh(),
    scratch_types=[
        pltpu.VMEM((CH, C), jnp.int32),
        pltpu.VMEM((C,), jnp.float32),
        pltpu.VMEM((ZSPAN,), jnp.float32),
        pltpu.VMEM_SHARED((ACC_ROWS,), jnp.float32),
    ],
)


def _hop_body(t_hbm, row_hbm, col_hbm, out_hbm, rowv, colv, rows_v, stage_v, acc, sem):
    c = lax.axis_index("c")
    s = lax.axis_index("s")
    w = c * NS + s

    def fill_z(i, carry):
        stage_v[i // (D // 16), pl.ds((i % (D // 16)) * 16, 16)] = jnp.zeros(
            (16,), jnp.float32
        )
        return carry

    lax.fori_loop(0, C * (D // 16), fill_z, 0)
    for q in range(ZSPAN // C):
        pltpu.sync_copy(stage_v, acc.at[pl.ds(s * ZSPAN + q * C, C)])
    plsc.subcore_barrier()

    for p in range(CH // CHP):
        pltpu.sync_copy(row_hbm.at[w, pl.ds(p * CHP, CHP)], rowv)
        pltpu.sync_copy(col_hbm.at[w, pl.ds(p * CHP, CHP)], colv)

        def chunk(j, carry):
            gather = pltpu.async_copy(t_hbm.at[rowv.at[j]], rows_v, sem)
            gather.wait()
            pltpu.sync_copy(rows_v, acc.at[colv.at[j]], add=True)
            return carry

        lax.fori_loop(0, CHP, chunk, 0)
    plsc.subcore_barrier()

    base = s * ZSPAN
    for q in range(ZSPAN // C):
        pltpu.sync_copy(acc.at[pl.ds(base + q * C, C)], stage_v)
        pltpu.sync_copy(stage_v, out_hbm.at[c, pl.ds(base + q * C, C)])


_hop_call = pl.kernel(
    _hop_body,
    out_type=jax.ShapeDtypeStruct((NC, ACC_ROWS, D), jnp.float32),
    mesh=_sc_mesh(),
    scratch_types=[
        pltpu.VMEM((CHP, C), jnp.int32),
        pltpu.VMEM((CHP, C), jnp.int32),
        pltpu.VMEM((C, D), jnp.float32),
        pltpu.VMEM((C, D), jnp.float32),
        pltpu.VMEM_SHARED((ACC_ROWS, D), jnp.float32),
        pltpu.SemaphoreType.DMA,
    ],
)


def _prep_body(degcol_ref, x_ref, t0_ref, sdeg_ref, rdeg_ref):
    deg = degcol_ref[:, 0:1] + degcol_ref[:, 1:2] + 1.0
    sdeg_ref[...] = lax.sqrt(deg)
    rdeg_ref[...] = 1.0 / deg
    t0_ref[...] = x_ref[...] * lax.rsqrt(deg)


def _combine_body(s_ref, t_ref, sc_ref, o_ref):
    o_ref[...] = (s_ref[0] + s_ref[1] + t_ref[...]) * sc_ref[...]


def _final_body(t_ref, sdeg_ref, wt_ref, b_ref, o_ref):
    h = t_ref[...] * sdeg_ref[...]
    o_ref[...] = (
        jnp.dot(h, wt_ref[...], preferred_element_type=jnp.float32) + b_ref[...]
    )


_RB = 2000
_G = N // _RB


def _tc_prep(degcol, x):
    return pl.pallas_call(
        _prep_body,
        grid=(_G,),
        in_specs=[
            pl.BlockSpec((_RB, 2), lambda i: (i, 0)),
            pl.BlockSpec((_RB, D), lambda i: (i, 0)),
        ],
        out_specs=[
            pl.BlockSpec((_RB, D), lambda i: (i, 0)),
            pl.BlockSpec((_RB, 1), lambda i: (i, 0)),
            pl.BlockSpec((_RB, 1), lambda i: (i, 0)),
        ],
        out_shape=[
            jax.ShapeDtypeStruct((N, D), jnp.float32),
            jax.ShapeDtypeStruct((N, 1), jnp.float32),
            jax.ShapeDtypeStruct((N, 1), jnp.float32),
        ],
    )(degcol, x)


def _tc_combine(s, t, scale):
    return pl.pallas_call(
        _combine_body,
        grid=(_G,),
        in_specs=[
            pl.BlockSpec((NC, _RB, D), lambda i: (0, i, 0)),
            pl.BlockSpec((_RB, D), lambda i: (i, 0)),
            pl.BlockSpec((_RB, 1), lambda i: (i, 0)),
        ],
        out_specs=pl.BlockSpec((_RB, D), lambda i: (i, 0)),
        out_shape=jax.ShapeDtypeStruct((N, D), jnp.float32),
    )(s, t, scale)


def _tc_final(t, sdeg, wt, b2):
    return pl.pallas_call(
        _final_body,
        grid=(_G,),
        in_specs=[
            pl.BlockSpec((_RB, D), lambda i: (i, 0)),
            pl.BlockSpec((_RB, 1), lambda i: (i, 0)),
            pl.BlockSpec((D, D), lambda i: (0, 0)),
            pl.BlockSpec((1, D), lambda i: (0, 0)),
        ],
        out_specs=pl.BlockSpec((_RB, D), lambda i: (i, 0)),
        out_shape=jax.ShapeDtypeStruct((N, D), jnp.float32),
    )(t, sdeg, wt, b2)


def kernel(x, edge_index, W, b):
    row = edge_index[0].astype(jnp.int32)
    col = edge_index[1].astype(jnp.int32)
    pad = EP - E
    padi = jnp.arange(pad, dtype=jnp.int32)
    row3 = jnp.concatenate([row, padi % N]).reshape(NW, CH, C)
    col3 = jnp.concatenate([col, N + padi % PAD_ROWS]).reshape(NW, CH, C)

    degp = _deg_call(col3)
    degcol = degp[:, :N].T
    t0, sdeg, rdeg = _tc_prep(degcol, x)

    def step(t, _):
        s = _hop_call(t, row3, col3)
        return _tc_combine(s, t, rdeg), None

    t2, _ = lax.scan(step, t0, None, length=2)

    return _tc_final(t2, sdeg, W.T, b.reshape(1, D))


# speedup vs baseline: 30.8799x; 1.4076x over previous
import functools

import jax
import jax.numpy as jnp
from jax import lax
from jax.experimental import pallas as pl
from jax.experimental.pallas import tpu as pltpu
from jax.experimental.pallas import tpu_sc as plsc

N = 10000
E = 320000
D = 128
NC = 2
NS = 16
NW = NC * NS
C = 128
CH = 80
CHP = 40
EPW = C * CH
EP = EPW * NW
PAD_ROWS = 240
ACC_ROWS = N + PAD_ROWS
ZSPAN = ACC_ROWS // NS
OSPAN = N // NS


def _sc_mesh():
    return plsc.VectorSubcoreMesh(core_axis_name="c", subcore_axis_name="s")


def _deg_body(col_hbm, out_hbm, colv, ones_v, zero_v, acc):
    c = lax.axis_index("c")
    s = lax.axis_index("s")
    w = c * NS + s

    def fill_z(i, carry):
        zero_v[pl.ds(i * 16, 16)] = jnp.zeros((16,), jnp.float32)
        return carry

    lax.fori_loop(0, ZSPAN // 16, fill_z, 0)

    def fill_o(i, carry):
        ones_v[pl.ds(i * 16, 16)] = jnp.ones((16,), jnp.float32)
        return carry

    lax.fori_loop(0, C // 16, fill_o, 0)

    pltpu.sync_copy(zero_v, acc.at[pl.ds(s * ZSPAN, ZSPAN)])
    plsc.subcore_barrier()

    pltpu.sync_copy(col_hbm.at[w], colv)

    def chunk(j, carry):
        pltpu.sync_copy(ones_v, acc.at[colv.at[j]], add=True)
        return carry

    lax.fori_loop(0, CH, chunk, 0)
    plsc.subcore_barrier()

    pltpu.sync_copy(acc.at[pl.ds(s * ZSPAN, ZSPAN)], zero_v)
    pltpu.sync_copy(zero_v, out_hbm.at[c, pl.ds(s * ZSPAN, ZSPAN)])


_deg_call = pl.kernel(
    _deg_body,
    out_type=jax.ShapeDtypeStruct((NC, ACC_ROWS), jnp.float32),
    mesh=_sc_mesh(),
    scratch_types=[
        pltpu.VMEM((CH, C), jnp.int32),
        pltpu.VMEM((C,), jnp.float32),
        pltpu.VMEM((ZSPAN,), jnp.float32),
        pltpu.VMEM_SHARED((ACC_ROWS,), jnp.float32),
    ],
)


def _hop_body(t_hbm, z_hbm, row_hbm, col_hbm, out_hbm, rowv, colv, bufa, bufb, acc, sema, semb):
    c = lax.axis_index("c")
    s = lax.axis_index("s")
    w = c * NS + s

    pltpu.sync_copy(z_hbm.at[pl.ds(s * ZSPAN, ZSPAN)], acc.at[pl.ds(s * ZSPAN, ZSPAN)])
    plsc.subcore_barrier()

    for p in range(CH // CHP):
        pltpu.sync_copy(row_hbm.at[w, pl.ds(p * CHP, CHP)], rowv)
        pltpu.sync_copy(col_hbm.at[w, pl.ds(p * CHP, CHP)], colv)

        pltpu.async_copy(t_hbm.at[rowv.at[0]], bufa, sema)

        def pipe(i, carry):
            j = 2 * i
            pltpu.async_copy(t_hbm.at[rowv.at[j + 1]], bufb, semb)
            pltpu.make_async_copy(t_hbm.at[pl.ds(0, C)], bufa, sema).wait()
            pltpu.sync_copy(bufa, acc.at[colv.at[j]], add=True)

            @pl.when(j + 2 < CHP)
            def _():
                pltpu.async_copy(t_hbm.at[rowv.at[j + 2]], bufa, sema)

            pltpu.make_async_copy(t_hbm.at[pl.ds(0, C)], bufb, semb).wait()
            pltpu.sync_copy(bufb, acc.at[colv.at[j + 1]], add=True)
            return carry

        lax.fori_loop(0, CHP // 2, pipe, 0)
    plsc.subcore_barrier()

    pltpu.sync_copy(
        acc.at[pl.ds(s * ZSPAN, ZSPAN)], out_hbm.at[c, pl.ds(s * ZSPAN, ZSPAN)]
    )


_hop_call = pl.kernel(
    _hop_body,
    out_type=jax.ShapeDtypeStruct((NC, ACC_ROWS, D), jnp.float32),
    mesh=_sc_mesh(),
    scratch_types=[
        pltpu.VMEM((CHP, C), jnp.int32),
        pltpu.VMEM((CHP, C), jnp.int32),
        pltpu.VMEM((C, D), jnp.float32),
        pltpu.VMEM((C, D), jnp.float32),
        pltpu.VMEM_SHARED((ACC_ROWS, D), jnp.float32),
        pltpu.SemaphoreType.DMA,
        pltpu.SemaphoreType.DMA,
    ],
)


def _prep_body(degcol_ref, x_ref, t0_ref, sdeg_ref, rdeg_ref):
    deg = degcol_ref[:, 0:1] + degcol_ref[:, 1:2] + 1.0
    sdeg_ref[...] = lax.sqrt(deg)
    rdeg_ref[...] = 1.0 / deg
    t0_ref[...] = x_ref[...] * lax.rsqrt(deg)


def _combine_body(s_ref, t_ref, sc_ref, o_ref):
    o_ref[...] = (s_ref[0] + s_ref[1] + t_ref[...]) * sc_ref[...]


def _final_body(t_ref, sdeg_ref, wt_ref, b_ref, o_ref):
    h = t_ref[...] * sdeg_ref[...]
    o_ref[...] = (
        jnp.dot(h, wt_ref[...], preferred_element_type=jnp.float32) + b_ref[...]
    )


_RB = 2000
_G = N // _RB


def _tc_prep(degcol, x):
    return pl.pallas_call(
        _prep_body,
        grid=(_G,),
        in_specs=[
            pl.BlockSpec((_RB, 2), lambda i: (i, 0)),
            pl.BlockSpec((_RB, D), lambda i: (i, 0)),
        ],
        out_specs=[
            pl.BlockSpec((_RB, D), lambda i: (i, 0)),
            pl.BlockSpec((_RB, 1), lambda i: (i, 0)),
            pl.BlockSpec((_RB, 1), lambda i: (i, 0)),
        ],
        out_shape=[
            jax.ShapeDtypeStruct((N, D), jnp.float32),
            jax.ShapeDtypeStruct((N, 1), jnp.float32),
            jax.ShapeDtypeStruct((N, 1), jnp.float32),
        ],
    )(degcol, x)


def _tc_combine(s, t, scale):
    return pl.pallas_call(
        _combine_body,
        grid=(_G,),
        in_specs=[
            pl.BlockSpec((NC, _RB, D), lambda i: (0, i, 0)),
            pl.BlockSpec((_RB, D), lambda i: (i, 0)),
            pl.BlockSpec((_RB, 1), lambda i: (i, 0)),
        ],
        out_specs=pl.BlockSpec((_RB, D), lambda i: (i, 0)),
        out_shape=jax.ShapeDtypeStruct((N, D), jnp.float32),
    )(s, t, scale)


def _tc_final(t, sdeg, wt, b2):
    return pl.pallas_call(
        _final_body,
        grid=(_G,),
        in_specs=[
            pl.BlockSpec((_RB, D), lambda i: (i, 0)),
            pl.BlockSpec((_RB, 1), lambda i: (i, 0)),
            pl.BlockSpec((D, D), lambda i: (0, 0)),
            pl.BlockSpec((1, D), lambda i: (0, 0)),
        ],
        out_specs=pl.BlockSpec((_RB, D), lambda i: (i, 0)),
        out_shape=jax.ShapeDtypeStruct((N, D), jnp.float32),
    )(t, sdeg, wt, b2)


def kernel(x, edge_index, W, b):
    row = edge_index[0].astype(jnp.int32)
    col = edge_index[1].astype(jnp.int32)
    pad = EP - E
    padi = jnp.arange(pad, dtype=jnp.int32)
    row3 = jnp.concatenate([row, padi % N]).reshape(NW, CH, C)
    col3 = jnp.concatenate([col, N + padi % PAD_ROWS]).reshape(NW, CH, C)

    degp = _deg_call(col3)
    degcol = degp[:, :N].T
    t0, sdeg, rdeg = _tc_prep(degcol, x)

    zeros = jnp.zeros((ACC_ROWS, D), jnp.float32)

    def step(t, _):
        s = _hop_call(t, zeros, row3, col3)
        return _tc_combine(s, t, rdeg), None

    t2, _ = lax.scan(step, t0, None, length=2)

    return _tc_final(t2, sdeg, W.T, b.reshape(1, D))


# speedup vs baseline: 32.8126x; 1.0626x over previous
import functools

import jax
import jax.numpy as jnp
from jax import lax
from jax.experimental import pallas as pl
from jax.experimental.pallas import tpu as pltpu
from jax.experimental.pallas import tpu_sc as plsc

N = 10000
E = 320000
D = 128
NC = 2
NS = 16
NW = NC * NS
C = 128
CH = 80
CHP = 40
EPW = C * CH
EP = EPW * NW
PAD_ROWS = 240
ACC_ROWS = N + PAD_ROWS
ZSPAN = ACC_ROWS // NS
OSPAN = N // NS


def _sc_mesh():
    return plsc.VectorSubcoreMesh(core_axis_name="c", subcore_axis_name="s")


def _deg_body(col_hbm, out_hbm, colv, ones_v, zero_v, acc):
    c = lax.axis_index("c")
    s = lax.axis_index("s")
    w = c * NS + s

    def fill_z(i, carry):
        zero_v[pl.ds(i * 16, 16)] = jnp.zeros((16,), jnp.float32)
        return carry

    lax.fori_loop(0, ZSPAN // 16, fill_z, 0)

    def fill_o(i, carry):
        ones_v[pl.ds(i * 16, 16)] = jnp.ones((16,), jnp.float32)
        return carry

    lax.fori_loop(0, C // 16, fill_o, 0)

    pltpu.sync_copy(zero_v, acc.at[pl.ds(s * ZSPAN, ZSPAN)])
    plsc.subcore_barrier()

    pltpu.sync_copy(col_hbm.at[w], colv)

    def chunk(j, carry):
        pltpu.sync_copy(ones_v, acc.at[colv.at[j]], add=True)
        return carry

    lax.fori_loop(0, CH, chunk, 0)
    plsc.subcore_barrier()

    pltpu.sync_copy(acc.at[pl.ds(s * ZSPAN, ZSPAN)], zero_v)
    pltpu.sync_copy(zero_v, out_hbm.at[c, pl.ds(s * ZSPAN, ZSPAN)])


_deg_call = pl.kernel(
    _deg_body,
    out_type=jax.ShapeDtypeStruct((NC, ACC_ROWS), jnp.float32),
    mesh=_sc_mesh(),
    scratch_types=[
        pltpu.VMEM((CH, C), jnp.int32),
        pltpu.VMEM((C,), jnp.float32),
        pltpu.VMEM((ZSPAN,), jnp.float32),
        pltpu.VMEM_SHARED((ACC_ROWS,), jnp.float32),
    ],
)


def _hop_body(t_hbm, z_hbm, row_hbm, col_hbm, out_hbm, rowv, colv, bufa, bufb, acc, sema, semb):
    c = lax.axis_index("c")
    s = lax.axis_index("s")
    w = c * NS + s

    pltpu.sync_copy(z_hbm.at[pl.ds(s * ZSPAN, ZSPAN)], acc.at[pl.ds(s * ZSPAN, ZSPAN)])
    plsc.subcore_barrier()

    for p in range(CH // CHP):
        pltpu.sync_copy(row_hbm.at[w, pl.ds(p * CHP, CHP)], rowv)
        pltpu.sync_copy(col_hbm.at[w, pl.ds(p * CHP, CHP)], colv)

        pltpu.async_copy(t_hbm.at[rowv.at[0]], bufa, sema)

        def pipe(i, carry):
            j = 2 * i
            pltpu.async_copy(t_hbm.at[rowv.at[j + 1]], bufb, semb)
            pltpu.make_async_copy(t_hbm.at[pl.ds(0, C)], bufa, sema).wait()
            pltpu.sync_copy(bufa, acc.at[colv.at[j]], add=True)

            @pl.when(j + 2 < CHP)
            def _():
                pltpu.async_copy(t_hbm.at[rowv.at[j + 2]], bufa, sema)

            pltpu.make_async_copy(t_hbm.at[pl.ds(0, C)], bufb, semb).wait()
            pltpu.sync_copy(bufb, acc.at[colv.at[j + 1]], add=True)
            return carry

        lax.fori_loop(0, CHP // 2, pipe, 0)
    plsc.subcore_barrier()

    pltpu.sync_copy(
        acc.at[pl.ds(s * ZSPAN, ZSPAN)], out_hbm.at[c, pl.ds(s * ZSPAN, ZSPAN)]
    )


_hop_call = pl.kernel(
    _hop_body,
    out_type=jax.ShapeDtypeStruct((NC, ACC_ROWS, D), jnp.float32),
    mesh=_sc_mesh(),
    scratch_types=[
        pltpu.VMEM((CHP, C), jnp.int32),
        pltpu.VMEM((CHP, C), jnp.int32),
        pltpu.VMEM((C, D), jnp.float32),
        pltpu.VMEM((C, D), jnp.float32),
        pltpu.VMEM_SHARED((ACC_ROWS, D), jnp.float32),
        pltpu.SemaphoreType.DMA,
        pltpu.SemaphoreType.DMA,
    ],
)


def _prep_body(degcol_ref, x_ref, t0_ref, dinv_ref, rdeg_ref):
    deg = degcol_ref[:, 0:1] + degcol_ref[:, 1:2] + 1.0
    dinv = lax.rsqrt(deg)
    dinv_ref[...] = dinv
    rdeg_ref[...] = 1.0 / deg
    t0_ref[...] = x_ref[...] * dinv


def _combine_body(s_ref, t_ref, sc_ref, o_ref):
    o_ref[...] = (s_ref[0] + s_ref[1] + t_ref[...]) * sc_ref[...]


def _final_body(s_ref, t_ref, dinv_ref, wt_ref, b_ref, o_ref):
    h = (s_ref[0] + s_ref[1] + t_ref[...]) * dinv_ref[...]
    o_ref[...] = (
        jnp.dot(h, wt_ref[...], preferred_element_type=jnp.float32) + b_ref[...]
    )


_RB = 2000
_G = N // _RB


def _tc_prep(degcol, x):
    return pl.pallas_call(
        _prep_body,
        grid=(_G,),
        in_specs=[
            pl.BlockSpec((_RB, 2), lambda i: (i, 0)),
            pl.BlockSpec((_RB, D), lambda i: (i, 0)),
        ],
        out_specs=[
            pl.BlockSpec((_RB, D), lambda i: (i, 0)),
            pl.BlockSpec((_RB, 1), lambda i: (i, 0)),
            pl.BlockSpec((_RB, 1), lambda i: (i, 0)),
        ],
        out_shape=[
            jax.ShapeDtypeStruct((N, D), jnp.float32),
            jax.ShapeDtypeStruct((N, 1), jnp.float32),
            jax.ShapeDtypeStruct((N, 1), jnp.float32),
        ],
    )(degcol, x)


def _tc_combine(s, t, scale):
    return pl.pallas_call(
        _combine_body,
        grid=(_G,),
        in_specs=[
            pl.BlockSpec((NC, _RB, D), lambda i: (0, i, 0)),
            pl.BlockSpec((_RB, D), lambda i: (i, 0)),
            pl.BlockSpec((_RB, 1), lambda i: (i, 0)),
        ],
        out_specs=pl.BlockSpec((_RB, D), lambda i: (i, 0)),
        out_shape=jax.ShapeDtypeStruct((N, D), jnp.float32),
    )(s, t, scale)


def _tc_final(s, t, dinv, wt, b2):
    return pl.pallas_call(
        _final_body,
        grid=(_G,),
        in_specs=[
            pl.BlockSpec((NC, _RB, D), lambda i: (0, i, 0)),
            pl.BlockSpec((_RB, D), lambda i: (i, 0)),
            pl.BlockSpec((_RB, 1), lambda i: (i, 0)),
            pl.BlockSpec((D, D), lambda i: (0, 0)),
            pl.BlockSpec((1, D), lambda i: (0, 0)),
        ],
        out_specs=pl.BlockSpec((_RB, D), lambda i: (i, 0)),
        out_shape=jax.ShapeDtypeStruct((N, D), jnp.float32),
    )(s, t, dinv, wt, b2)


def kernel(x, edge_index, W, b):
    row = edge_index[0].astype(jnp.int32)
    col = edge_index[1].astype(jnp.int32)
    pad = EP - E
    padi = jnp.arange(pad, dtype=jnp.int32)
    row3 = jnp.concatenate([row, padi % N]).reshape(NW, CH, C)
    col3 = jnp.concatenate([col, N + padi % PAD_ROWS]).reshape(NW, CH, C)

    degp = _deg_call(col3)
    degcol = degp[:, :N].T
    t0, dinv, rdeg = _tc_prep(degcol, x)

    zeros = jnp.zeros((ACC_ROWS, D), jnp.float32)
    s1 = _hop_call(t0, zeros, row3, col3)
    t1 = _tc_combine(s1, t0, rdeg)
    s2 = _hop_call(t1, zeros, row3, col3)
    return _tc_final(s2, t1, dinv, W.T, b.reshape(1, D))


# speedup vs baseline: 32.9794x; 1.0051x over previous
import jax
import jax.numpy as jnp
from jax import lax
from jax.experimental import pallas as pl
from jax.experimental.pallas import tpu as pltpu
from jax.experimental.pallas import tpu_sc as plsc

N = 10000
E = 320000
D = 128
NC = 2
NS = 16
NW = NC * NS
C = 128
CT = E // C
PH = 40
ACC_ROWS = 10240
ZSPAN = ACC_ROWS // NS
BASE30 = 80 * 30
BASE31 = BASE30 + 56
TAIL = BASE31 + 40
NTAIL = CT - TAIL


def _sc_mesh():
    return plsc.VectorSubcoreMesh(core_axis_name="c", subcore_axis_name="s")


def _worker_range(w):
    base = jnp.where(w < 30, 80 * w, jnp.where(w == 30, BASE30, BASE31))
    cnt = jnp.where(w < 30, 80, jnp.where(w == 30, 56, 40))
    return base, cnt


def _deg_body(col_hbm, out_hbm, colv, ones_v, zero_v, acc):
    c = lax.axis_index("c")
    s = lax.axis_index("s")
    w = c * NS + s
    base, cnt = _worker_range(w)

    def fill_z(i, carry):
        zero_v[pl.ds(i * 16, 16)] = jnp.zeros((16,), jnp.float32)
        return carry

    lax.fori_loop(0, ZSPAN // 16, fill_z, 0)

    def fill_o(i, carry):
        ones_v[pl.ds(i * 16, 16)] = jnp.ones((16,), jnp.float32)
        return carry

    lax.fori_loop(0, C // 16, fill_o, 0)

    pltpu.sync_copy(zero_v, acc.at[pl.ds(s * ZSPAN, ZSPAN)])
    plsc.subcore_barrier()

    def chunk(j, carry):
        pltpu.sync_copy(ones_v, acc.at[colv.at[j]], add=True)
        return carry

    pltpu.sync_copy(col_hbm.at[pl.ds(base, PH)], colv)
    lax.fori_loop(0, PH, chunk, 0)
    pltpu.sync_copy(col_hbm.at[pl.ds(base + cnt - PH, PH)], colv)
    lax.fori_loop(2 * PH - cnt, PH, chunk, 0)

    @pl.when(w == 31)
    def _():
        pltpu.sync_copy(col_hbm.at[pl.ds(TAIL, NTAIL)], colv.at[pl.ds(0, NTAIL)])
        lax.fori_loop(0, NTAIL, chunk, 0)

    plsc.subcore_barrier()

    pltpu.sync_copy(acc.at[pl.ds(s * ZSPAN, ZSPAN)], zero_v)
    pltpu.sync_copy(zero_v, out_hbm.at[c, pl.ds(s * ZSPAN, ZSPAN)])


_deg_call = pl.kernel(
    _deg_body,
    out_type=jax.ShapeDtypeStruct((NC, ACC_ROWS), jnp.float32),
    mesh=_sc_mesh(),
    scratch_types=[
        pltpu.VMEM((PH, C), jnp.int32),
        pltpu.VMEM((C,), jnp.float32),
        pltpu.VMEM((ZSPAN,), jnp.float32),
        pltpu.VMEM_SHARED((ACC_ROWS,), jnp.float32),
    ],
)


def _hop_body(t_hbm, z_hbm, row_hbm, col_hbm, out_hbm, rowv, colv, bufa, bufb, acc, sema, semb):
    c = lax.axis_index("c")
    s = lax.axis_index("s")
    w = c * NS + s
    base, cnt = _worker_range(w)

    pltpu.sync_copy(z_hbm.at[pl.ds(s * ZSPAN, ZSPAN)], acc.at[pl.ds(s * ZSPAN, ZSPAN)])
    plsc.subcore_barrier()

    def run_window(sw, j0):
        pltpu.sync_copy(row_hbm.at[pl.ds(sw, PH)], rowv)
        pltpu.sync_copy(col_hbm.at[pl.ds(sw, PH)], colv)

        @pl.when(j0 < PH)
        def _():
            pltpu.async_copy(t_hbm.at[rowv.at[j0]], bufa, sema)

        def pipe(i, carry):
            j = 2 * i
            pltpu.async_copy(t_hbm.at[rowv.at[j + 1]], bufb, semb)
            pltpu.make_async_copy(t_hbm.at[pl.ds(0, C)], bufa, sema).wait()
            pltpu.sync_copy(bufa, acc.at[colv.at[j]], add=True)

            @pl.when(j + 2 < PH)
            def _():
                pltpu.async_copy(t_hbm.at[rowv.at[j + 2]], bufa, sema)

            pltpu.make_async_copy(t_hbm.at[pl.ds(0, C)], bufb, semb).wait()
            pltpu.sync_copy(bufb, acc.at[colv.at[j + 1]], add=True)
            return carry

        lax.fori_loop(j0 // 2, PH // 2, pipe, 0)

    run_window(base, jnp.int32(0))
    run_window(base + cnt - PH, 2 * PH - cnt)

    @pl.when(w == 31)
    def _():
        pltpu.sync_copy(row_hbm.at[pl.ds(TAIL, NTAIL)], rowv.at[pl.ds(0, NTAIL)])
        pltpu.sync_copy(col_hbm.at[pl.ds(TAIL, NTAIL)], colv.at[pl.ds(0, NTAIL)])

        def tchunk(j, carry):
            pltpu.async_copy(t_hbm.at[rowv.at[j]], bufa, sema).wait()
            pltpu.sync_copy(bufa, acc.at[colv.at[j]], add=True)
            return carry

        lax.fori_loop(0, NTAIL, tchunk, 0)

    plsc.subcore_barrier()

    pltpu.sync_copy(
        acc.at[pl.ds(s * ZSPAN, ZSPAN)], out_hbm.at[c, pl.ds(s * ZSPAN, ZSPAN)]
    )


_hop_call = pl.kernel(
    _hop_body,
    out_type=jax.ShapeDtypeStruct((NC, ACC_ROWS, D), jnp.float32),
    mesh=_sc_mesh(),
    scratch_types=[
        pltpu.VMEM((PH, C), jnp.int32),
        pltpu.VMEM((PH, C), jnp.int32),
        pltpu.VMEM((C, D), jnp.float32),
        pltpu.VMEM((C, D), jnp.float32),
        pltpu.VMEM_SHARED((ACC_ROWS, D), jnp.float32),
        pltpu.SemaphoreType.DMA,
        pltpu.SemaphoreType.DMA,
    ],
)


def _prep_body(degcol_ref, x_ref, t0_ref, dinv_ref, rdeg_ref):
    deg = degcol_ref[:, 0:1] + degcol_ref[:, 1:2] + 1.0
    dinv = lax.rsqrt(deg)
    dinv_ref[...] = dinv
    rdeg_ref[...] = 1.0 / deg
    t0_ref[...] = x_ref[...] * dinv


def _combine_body(s_ref, t_ref, sc_ref, o_ref):
    o_ref[...] = (s_ref[0] + s_ref[1] + t_ref[...]) * sc_ref[...]


def _final_body(s_ref, t_ref, dinv_ref, wt_ref, b_ref, o_ref):
    h = (s_ref[0] + s_ref[1] + t_ref[...]) * dinv_ref[...]
    o_ref[...] = (
        jnp.dot(h, wt_ref[...], preferred_element_type=jnp.float32) + b_ref[...]
    )


_RB = 2000
_G = N // _RB


def _tc_prep(degcol, x):
    return pl.pallas_call(
        _prep_body,
        grid=(_G,),
        in_specs=[
            pl.BlockSpec((_RB, 2), lambda i: (i, 0)),
            pl.BlockSpec((_RB, D), lambda i: (i, 0)),
        ],
        out_specs=[
            pl.BlockSpec((_RB, D), lambda i: (i, 0)),
            pl.BlockSpec((_RB, 1), lambda i: (i, 0)),
            pl.BlockSpec((_RB, 1), lambda i: (i, 0)),
        ],
        out_shape=[
            jax.ShapeDtypeStruct((N, D), jnp.float32),
            jax.ShapeDtypeStruct((N, 1), jnp.float32),
            jax.ShapeDtypeStruct((N, 1), jnp.float32),
        ],
    )(degcol, x)


def _tc_combine(s, t, scale):
    return pl.pallas_call(
        _combine_body,
        grid=(_G,),
        in_specs=[
            pl.BlockSpec((NC, _RB, D), lambda i: (0, i, 0)),
            pl.BlockSpec((_RB, D), lambda i: (i, 0)),
            pl.BlockSpec((_RB, 1), lambda i: (i, 0)),
        ],
        out_specs=pl.BlockSpec((_RB, D), lambda i: (i, 0)),
        out_shape=jax.ShapeDtypeStruct((N, D), jnp.float32),
    )(s, t, scale)


def _tc_final(s, t, dinv, wt, b2):
    return pl.pallas_call(
        _final_body,
        grid=(_G,),
        in_specs=[
            pl.BlockSpec((NC, _RB, D), lambda i: (0, i, 0)),
            pl.BlockSpec((_RB, D), lambda i: (i, 0)),
            pl.BlockSpec((_RB, 1), lambda i: (i, 0)),
            pl.BlockSpec((D, D), lambda i: (0, 0)),
            pl.BlockSpec((1, D), lambda i: (0, 0)),
        ],
        out_specs=pl.BlockSpec((_RB, D), lambda i: (i, 0)),
        out_shape=jax.ShapeDtypeStruct((N, D), jnp.float32),
    )(s, t, dinv, wt, b2)


def kernel(x, edge_index, W, b):
    row2 = edge_index[0].reshape(CT, C)
    col2 = edge_index[1].reshape(CT, C)

    degp = _deg_call(col2)
    degcol = degp[:, :N].T
    t0, dinv, rdeg = _tc_prep(degcol, x)

    zeros = jnp.zeros((ACC_ROWS, D), jnp.float32)
    s1 = _hop_call(t0, zeros, row2, col2)
    t1 = _tc_combine(s1, t0, rdeg)
    s2 = _hop_call(t1, zeros, row2, col2)
    return _tc_final(s2, t1, dinv, W.T, b.reshape(1, D))


# speedup vs baseline: 33.9366x; 1.0290x over previous
import jax
import jax.numpy as jnp
from jax import lax
from jax.experimental import pallas as pl
from jax.experimental.pallas import tpu as pltpu
from jax.experimental.pallas import tpu_sc as plsc

N = 10000
E = 320000
D = 128
NC = 2
NS = 16
NW = NC * NS
C = 128
CT = E // C
PH = 40
ACC_ROWS = 10240
ZSPAN = ACC_ROWS // NS
BASE30 = 80 * 30
BASE31 = BASE30 + 56
TAIL = BASE31 + 40
NTAIL = CT - TAIL


def _sc_mesh():
    return plsc.VectorSubcoreMesh(core_axis_name="c", subcore_axis_name="s")


def _worker_range(w):
    base = jnp.where(w < 30, 80 * w, jnp.where(w == 30, BASE30, BASE31))
    cnt = jnp.where(w < 30, 80, jnp.where(w == 30, 56, 40))
    return base, cnt


def _deg_body(e_hbm, out_hbm, colv, ones_v, zero_v, acc):
    c = lax.axis_index("c")
    s = lax.axis_index("s")
    w = c * NS + s
    base, cnt = _worker_range(w)

    def fill_z(i, carry):
        zero_v[pl.ds(i * 16, 16)] = jnp.zeros((16,), jnp.float32)
        return carry

    lax.fori_loop(0, ZSPAN // 16, fill_z, 0)

    def fill_o(i, carry):
        ones_v[pl.ds(i * 16, 16)] = jnp.ones((16,), jnp.float32)
        return carry

    lax.fori_loop(0, C // 16, fill_o, 0)

    pltpu.sync_copy(zero_v, acc.at[pl.ds(s * ZSPAN, ZSPAN)])
    plsc.subcore_barrier()

    def chunk(j, carry):
        pltpu.sync_copy(ones_v, acc.at[colv.at[j]], add=True)
        return carry

    pltpu.sync_copy(e_hbm.at[1, pl.ds(base, PH)], colv)
    lax.fori_loop(0, PH, chunk, 0)
    pltpu.sync_copy(e_hbm.at[1, pl.ds(base + cnt - PH, PH)], colv)
    lax.fori_loop(2 * PH - cnt, PH, chunk, 0)

    @pl.when(w == 31)
    def _():
        pltpu.sync_copy(e_hbm.at[1, pl.ds(TAIL, NTAIL)], colv.at[pl.ds(0, NTAIL)])
        lax.fori_loop(0, NTAIL, chunk, 0)

    plsc.subcore_barrier()

    pltpu.sync_copy(acc.at[pl.ds(s * ZSPAN, ZSPAN)], zero_v)
    pltpu.sync_copy(zero_v, out_hbm.at[c, pl.ds(s * ZSPAN, ZSPAN)])


_deg_call = pl.kernel(
    _deg_body,
    out_type=jax.ShapeDtypeStruct((NC, ACC_ROWS), jnp.float32),
    mesh=_sc_mesh(),
    scratch_types=[
        pltpu.VMEM((PH, C), jnp.int32),
        pltpu.VMEM((C,), jnp.float32),
        pltpu.VMEM((ZSPAN,), jnp.float32),
        pltpu.VMEM_SHARED((ACC_ROWS,), jnp.float32),
    ],
)


def _hop_body(t_hbm, z_hbm, e_hbm, out_hbm, rowv, colv, bufa, bufb, acc, sema, semb):
    c = lax.axis_index("c")
    s = lax.axis_index("s")
    w = c * NS + s
    base, cnt = _worker_range(w)

    pltpu.sync_copy(z_hbm.at[pl.ds(s * ZSPAN, ZSPAN)], acc.at[pl.ds(s * ZSPAN, ZSPAN)])
    plsc.subcore_barrier()

    def run_window(sw, j0):
        pltpu.sync_copy(e_hbm.at[0, pl.ds(sw, PH)], rowv)
        pltpu.sync_copy(e_hbm.at[1, pl.ds(sw, PH)], colv)

        @pl.when(j0 < PH)
        def _():
            pltpu.async_copy(t_hbm.at[rowv.at[j0]], bufa, sema)

        def pipe(i, carry):
            j = 2 * i
            pltpu.async_copy(t_hbm.at[rowv.at[j + 1]], bufb, semb)
            pltpu.make_async_copy(t_hbm.at[pl.ds(0, C)], bufa, sema).wait()
            pltpu.sync_copy(bufa, acc.at[colv.at[j]], add=True)

            @pl.when(j + 2 < PH)
            def _():
                pltpu.async_copy(t_hbm.at[rowv.at[j + 2]], bufa, sema)

            pltpu.make_async_copy(t_hbm.at[pl.ds(0, C)], bufb, semb).wait()
            pltpu.sync_copy(bufb, acc.at[colv.at[j + 1]], add=True)
            return carry

        lax.fori_loop(j0 // 2, PH // 2, pipe, 0)

    run_window(base, jnp.int32(0))
    run_window(base + cnt - PH, 2 * PH - cnt)

    @pl.when(w == 31)
    def _():
        pltpu.sync_copy(e_hbm.at[0, pl.ds(TAIL, NTAIL)], rowv.at[pl.ds(0, NTAIL)])
        pltpu.sync_copy(e_hbm.at[1, pl.ds(TAIL, NTAIL)], colv.at[pl.ds(0, NTAIL)])

        def tchunk(j, carry):
            pltpu.async_copy(t_hbm.at[rowv.at[j]], bufa, sema).wait()
            pltpu.sync_copy(bufa, acc.at[colv.at[j]], add=True)
            return carry

        lax.fori_loop(0, NTAIL, tchunk, 0)

    plsc.subcore_barrier()

    pltpu.sync_copy(
        acc.at[pl.ds(s * ZSPAN, ZSPAN)], out_hbm.at[c, pl.ds(s * ZSPAN, ZSPAN)]
    )


_hop_call = pl.kernel(
    _hop_body,
    out_type=jax.ShapeDtypeStruct((NC, ACC_ROWS, D), jnp.float32),
    mesh=_sc_mesh(),
    scratch_types=[
        pltpu.VMEM((PH, C), jnp.int32),
        pltpu.VMEM((PH, C), jnp.int32),
        pltpu.VMEM((C, D), jnp.float32),
        pltpu.VMEM((C, D), jnp.float32),
        pltpu.VMEM_SHARED((ACC_ROWS, D), jnp.float32),
        pltpu.SemaphoreType.DMA,
        pltpu.SemaphoreType.DMA,
    ],
)


def _prep_body(degcol_ref, x_ref, t0_ref, dinv_ref, rdeg_ref):
    deg = degcol_ref[:, 0:1] + degcol_ref[:, 1:2] + 1.0
    dinv = lax.rsqrt(deg)
    dinv_ref[...] = dinv
    rdeg_ref[...] = 1.0 / deg
    t0_ref[...] = x_ref[...] * dinv


def _combine_body(s_ref, t_ref, sc_ref, o_ref):
    o_ref[...] = (s_ref[0] + s_ref[1] + t_ref[...]) * sc_ref[...]


def _final_body(s_ref, t_ref, dinv_ref, wt_ref, b_ref, o_ref):
    h = (s_ref[0] + s_ref[1] + t_ref[...]) * dinv_ref[...]
    o_ref[...] = (
        jnp.dot(h, wt_ref[...], preferred_element_type=jnp.float32) + b_ref[...]
    )


_RB = 2000
_G = N // _RB


def _tc_prep(degcol, x):
    return pl.pallas_call(
        _prep_body,
        grid=(_G,),
        in_specs=[
            pl.BlockSpec((_RB, 2), lambda i: (i, 0)),
            pl.BlockSpec((_RB, D), lambda i: (i, 0)),
        ],
        out_specs=[
            pl.BlockSpec((_RB, D), lambda i: (i, 0)),
            pl.BlockSpec((_RB, 1), lambda i: (i, 0)),
            pl.BlockSpec((_RB, 1), lambda i: (i, 0)),
        ],
        out_shape=[
            jax.ShapeDtypeStruct((N, D), jnp.float32),
            jax.ShapeDtypeStruct((N, 1), jnp.float32),
            jax.ShapeDtypeStruct((N, 1), jnp.float32),
        ],
    )(degcol, x)


def _tc_combine(s, t, scale):
    return pl.pallas_call(
        _combine_body,
        grid=(_G,),
        in_specs=[
            pl.BlockSpec((NC, _RB, D), lambda i: (0, i, 0)),
            pl.BlockSpec((_RB, D), lambda i: (i, 0)),
            pl.BlockSpec((_RB, 1), lambda i: (i, 0)),
        ],
        out_specs=pl.BlockSpec((_RB, D), lambda i: (i, 0)),
        out_shape=jax.ShapeDtypeStruct((N, D), jnp.float32),
    )(s, t, scale)


def _tc_final(s, t, dinv, wt, b2):
    return pl.pallas_call(
        _final_body,
        grid=(_G,),
        in_specs=[
            pl.BlockSpec((NC, _RB, D), lambda i: (0, i, 0)),
            pl.BlockSpec((_RB, D), lambda i: (i, 0)),
            pl.BlockSpec((_RB, 1), lambda i: (i, 0)),
            pl.BlockSpec((D, D), lambda i: (0, 0)),
            pl.BlockSpec((1, D), lambda i: (0, 0)),
        ],
        out_specs=pl.BlockSpec((_RB, D), lambda i: (i, 0)),
        out_shape=jax.ShapeDtypeStruct((N, D), jnp.float32),
    )(s, t, dinv, wt, b2)


def kernel(x, edge_index, W, b):
    e3 = edge_index.reshape(2, CT, C)

    degp = _deg_call(e3)
    degcol = degp[:, :N].T
    t0, dinv, rdeg = _tc_prep(degcol, x)

    zeros = jnp.zeros((ACC_ROWS, D), jnp.float32)
    s1 = _hop_call(t0, zeros, e3)
    t1 = _tc_combine(s1, t0, rdeg)
    s2 = _hop_call(t1, zeros, e3)
    return _tc_final(s2, t1, dinv, W.T, b.reshape(1, D))
